# Initial kernel scaffold; baseline (speedup 1.0000x reference)
import jax
import jax.numpy as jnp
from jax.experimental import pallas as pl


def kernel(z, codebook):
    raise NotImplementedError("write your pallas kernel here")


# speedup vs baseline: 1.3714x; 1.3714x over previous
import functools

import jax
import jax.numpy as jnp
from jax.experimental import pallas as pl
from jax.experimental.pallas import tpu as pltpu

NUM_EMBEDDINGS = 512
EMBEDDING_DIM = 32
COMMITMENT_COST = 0.25
BLOCK_TOKENS = 2048


def _vq_block_kernel(x_ref, cb_ref, q_ref, idx_ref, com_ref, cbl_ref, tot_ref,
                     acc_ref, *, n_total, n_blocks):
    i = pl.program_id(0)
    x = x_ref[...]
    cb = cb_ref[...]

    def _rowsum32(s):
        t = s[:, 0:8] + s[:, 8:16]
        t = t + s[:, 16:24]
        t = t + s[:, 24:32]
        t = t[:, 0:4] + t[:, 4:8]
        t = t[:, 0:2] + t[:, 2:4]
        return t[:, 0:1] + t[:, 1:2]

    x2 = _rowsum32(x * x)
    c2 = _rowsum32(cb * cb).reshape(1, -1)
    xc = jax.lax.dot_general(
        x.astype(jnp.bfloat16), cb.astype(jnp.bfloat16),
        (((1,), (1,)), ((), ())),
        preferred_element_type=jnp.float32)
    d2 = jnp.clip(x2 - 2.0 * xc + c2, 0.0, None)
    dist = jnp.sqrt(d2)
    b, k = d2.shape
    min_dist = jnp.min(dist, axis=1, keepdims=True)
    iota = jax.lax.broadcasted_iota(jnp.int32, (b, k), 1)
    idx = jnp.min(jnp.where(dist == min_dist, iota, k), axis=1)
    idx = idx.astype(jnp.int32)
    onehot = (iota == idx[:, None]).astype(jnp.float32)
    q = jax.lax.dot_general(
        onehot, cb, (((1,), (0,)), ((), ())),
        preferred_element_type=jnp.float32)
    q_ref[...] = q
    idx_ref[...] = idx.reshape(1, 1, b)
    diff = x - q
    part = jnp.sum(diff * diff)

    @pl.when(i == 0)
    def _():
        acc_ref[0, 0] = part

    @pl.when(i > 0)
    def _():
        acc_ref[0, 0] += part

    @pl.when(i == n_blocks - 1)
    def _():
        m = acc_ref[0, 0] / n_total
        com_ref[...] = jnp.full((1, 1), m * COMMITMENT_COST, jnp.float32)
        cbl_ref[...] = jnp.full((1, 1), m, jnp.float32)
        tot_ref[...] = jnp.full((1, 1), m * (1.0 + COMMITMENT_COST),
                                jnp.float32)


def kernel(z, codebook):
    orig_shape = z.shape
    d = codebook.shape[1]
    x = z.reshape(-1, d)
    n = x.shape[0]
    b = BLOCK_TOKENS
    n_blocks = n // b
    k = codebook.shape[0]
    n_total = float(n * d)

    body = functools.partial(_vq_block_kernel, n_total=n_total,
                             n_blocks=n_blocks)
    scalar_spec = pl.BlockSpec((1, 1), lambda i: (0, 0))
    q, idx3, com, cbl, tot = pl.pallas_call(
        body,
        grid=(n_blocks,),
        in_specs=[
            pl.BlockSpec((b, d), lambda i: (i, 0)),
            pl.BlockSpec((k, d), lambda i: (0, 0)),
        ],
        out_specs=[
            pl.BlockSpec((b, d), lambda i: (i, 0)),
            pl.BlockSpec((1, 1, b), lambda i: (i, 0, 0)),
            scalar_spec, scalar_spec, scalar_spec,
        ],
        out_shape=[
            jax.ShapeDtypeStruct((n, d), jnp.float32),
            jax.ShapeDtypeStruct((n_blocks, 1, b), jnp.int32),
            jax.ShapeDtypeStruct((1, 1), jnp.float32),
            jax.ShapeDtypeStruct((1, 1), jnp.float32),
            jax.ShapeDtypeStruct((1, 1), jnp.float32),
        ],
        scratch_shapes=[pltpu.SMEM((1, 1), jnp.float32)],
    )(x, codebook)

    quantized = q.reshape(orig_shape)
    indices = idx3.reshape(n)
    return (quantized, indices, com.reshape(()), cbl.reshape(()),
            tot.reshape(()))
